# Initial kernel scaffold; baseline (speedup 1.0000x reference)
import functools

import jax
import jax.numpy as jnp
from jax.experimental import pallas as pl
from jax.experimental.pallas import tpu as pltpu

TEMP = 8.0
N = 4096
NTRK = 512
D = 32
R = 512
ACC_ROWS = 8


def _loss_kernel(xb_ref, x_ref, yf_ref, trk_ref, out_ref, acc_ref):
    i = pl.program_id(0)
    nsteps = pl.num_programs(0)

    @pl.when(i == 0)
    def _init():
        acc_ref[...] = jnp.zeros_like(acc_ref)

    xb = xb_ref[...]
    t_blk = trk_ref[pl.ds(i * R, R)]
    t_col = t_blk[:, None]

    dot = lambda a, b: jax.lax.dot_general(
        a, b, (((1,), (1,)), ((), ())), preferred_element_type=jnp.float32)

    S = jnp.exp(dot(xb, yf_ref[...]) * (1.0 / TEMP))
    Sx = jnp.exp(dot(xb, x_ref[...]) * (1.0 / TEMP))

    col = jax.lax.broadcasted_iota(jnp.int32, (R, N), 1)
    row_id = i * R + jax.lax.broadcasted_iota(jnp.int32, (R, N), 0)
    mask_s = (col & (NTRK - 1)) == t_col
    mask_sx = trk_ref[...][None, :] == t_col
    mask_diag = col == row_id

    zero = jnp.zeros_like(S)
    row_s = jnp.sum(S, axis=1)
    same_s = jnp.sum(jnp.where(mask_s, S, zero), axis=1)
    row_sx = jnp.sum(Sx, axis=1)
    same_sx = jnp.sum(jnp.where(mask_sx, Sx, zero), axis=1)
    self_sx = jnp.sum(jnp.where(mask_diag, Sx, zero), axis=1)

    a_num = same_s + 0.5 * (same_sx - self_sx)
    a_den = (row_s - same_s) + (row_sx - same_sx)

    hot = (t_col == jax.lax.broadcasted_iota(jnp.int32, (R, NTRK), 1)
           ).astype(jnp.float32)
    vals = jnp.concatenate(
        [a_num[None, :], a_den[None, :], jnp.ones((1, R), jnp.float32),
         jnp.zeros((ACC_ROWS - 3, R), jnp.float32)], axis=0)
    acc_ref[...] += jax.lax.dot_general(
        vals, hot, (((1,), (0,)), ((), ())),
        preferred_element_type=jnp.float32)

    @pl.when(i == nsteps - 1)
    def _finish():
        num = acc_ref[0, :]
        den = acc_ref[1, :]
        present = acc_ref[2, :] > 0.0
        safe_num = jnp.where(present, num, 1.0)
        safe_den = jnp.where(present, den, 1.0)
        per = jnp.where(
            present, -jnp.log(safe_num / (safe_den + safe_num)), 0.0)
        n_present = jnp.maximum(
            jnp.sum(present.astype(jnp.float32)), 1.0)
        out_ref[0] = jnp.sum(per) / n_present


@jax.jit
def _run(x, track_idxs, yf):
    nsteps = N // R
    return pl.pallas_call(
        _loss_kernel,
        grid=(nsteps,),
        in_specs=[
            pl.BlockSpec((R, D), lambda i: (i, 0)),
            pl.BlockSpec((N, D), lambda i: (0, 0)),
            pl.BlockSpec((N, D), lambda i: (0, 0)),
            pl.BlockSpec((N,), lambda i: (0,)),
        ],
        out_specs=pl.BlockSpec((1,), lambda i: (0,)),
        out_shape=jax.ShapeDtypeStruct((1,), jnp.float32),
        scratch_shapes=[pltpu.VMEM((ACC_ROWS, NTRK), jnp.float32)],
    )(x, x, yf, track_idxs)


def kernel(x, track_idxs, y):
    yf = y.reshape(-1, D)
    return _run(x, track_idxs.astype(jnp.int32), yf)


# speedup vs baseline: 2.3420x; 2.3420x over previous
import functools

import jax
import jax.numpy as jnp
from jax.experimental import pallas as pl
from jax.experimental.pallas import tpu as pltpu

TEMP = 8.0
N = 4096
NTRK = 512
D = 32
R = 512
ACC_ROWS = 8


def _loss_kernel(xb_ref, x_ref, yf_ref, trk_ref, out_ref, acc_ref):
    i = pl.program_id(0)
    nsteps = pl.num_programs(0)

    @pl.when(i == 0)
    def _init():
        acc_ref[...] = jnp.zeros_like(acc_ref)

    xb = xb_ref[...]
    t_blk = trk_ref[pl.ds(i * R, R)]
    t_col = t_blk[:, None]

    dot = lambda a, b: jax.lax.dot_general(
        a, b, (((1,), (1,)), ((), ())), preferred_element_type=jnp.float32)

    S = jnp.exp(dot(xb, yf_ref[...]) * (1.0 / TEMP))
    Sx = jnp.exp(dot(xb, x_ref[...]) * (1.0 / TEMP))

    col = jax.lax.broadcasted_iota(jnp.int32, (R, N), 1)
    row_id = i * R + jax.lax.broadcasted_iota(jnp.int32, (R, N), 0)
    mask_s = (col & (NTRK - 1)) == t_col
    mask_sx = trk_ref[...][None, :] == t_col
    mask_diag = col == row_id

    zero = jnp.zeros_like(S)
    row_s = jnp.sum(S, axis=1)
    same_s = jnp.sum(jnp.where(mask_s, S, zero), axis=1)
    row_sx = jnp.sum(Sx, axis=1)
    same_sx = jnp.sum(jnp.where(mask_sx, Sx, zero), axis=1)
    self_sx = jnp.sum(jnp.where(mask_diag, Sx, zero), axis=1)

    a_num = same_s + 0.5 * (same_sx - self_sx)
    a_den = (row_s - same_s) + (row_sx - same_sx)

    hot = (t_col == jax.lax.broadcasted_iota(jnp.int32, (R, NTRK), 1)
           ).astype(jnp.float32)
    vals = jnp.concatenate(
        [a_num[None, :], a_den[None, :], jnp.ones((1, R), jnp.float32),
         jnp.zeros((ACC_ROWS - 3, R), jnp.float32)], axis=0)
    acc_ref[...] += jax.lax.dot_general(
        vals, hot, (((1,), (0,)), ((), ())),
        preferred_element_type=jnp.float32)

    @pl.when(i == nsteps - 1)
    def _finish():
        num = acc_ref[0, :]
        den = acc_ref[1, :]
        present = acc_ref[2, :] > 0.0
        safe_num = jnp.where(present, num, 1.0)
        safe_den = jnp.where(present, den, 1.0)
        per = jnp.where(
            present, -jnp.log(safe_num / (safe_den + safe_num)), 0.0)
        n_present = jnp.maximum(
            jnp.sum(present.astype(jnp.float32)), 1.0)
        out_ref[...] = (jnp.sum(per) / n_present).reshape(1)


@jax.jit
def _run(x, track_idxs, yf):
    nsteps = N // R
    return pl.pallas_call(
        _loss_kernel,
        grid=(nsteps,),
        in_specs=[
            pl.BlockSpec((R, D), lambda i: (i, 0)),
            pl.BlockSpec((N, D), lambda i: (0, 0)),
            pl.BlockSpec((N, D), lambda i: (0, 0)),
            pl.BlockSpec((N,), lambda i: (0,)),
        ],
        out_specs=pl.BlockSpec((1,), lambda i: (0,)),
        out_shape=jax.ShapeDtypeStruct((1,), jnp.float32),
        scratch_shapes=[pltpu.VMEM((ACC_ROWS, NTRK), jnp.float32)],
    )(x, x, yf, track_idxs)


def kernel(x, track_idxs, y):
    yf = y.reshape(-1, D)
    return _run(x, track_idxs.astype(jnp.int32), yf)


# speedup vs baseline: 2.6591x; 1.1354x over previous
import jax
import jax.numpy as jnp
from jax.experimental import pallas as pl
from jax.experimental.pallas import tpu as pltpu

TEMP = 8.0
LN2 = 0.6931471805599453
N = 4096
NTRK = 512
D = 32
DA = D + 1
W = 2 * N
R = 512
ACC_ROWS = 8


def _loss_kernel(xb_ref, comb_ref, colid_ref, out_ref, acc_ref):
    i = pl.program_id(0)
    nsteps = pl.num_programs(0)

    @pl.when(i == 0)
    def _init():
        acc_ref[...] = jnp.zeros_like(acc_ref)

    fcol = jax.lax.broadcasted_iota(jnp.int32, (R, DA), 1)
    xb = jnp.where(fcol < D, xb_ref[...] * (1.0 / TEMP), xb_ref[...])

    Z = jax.lax.dot_general(
        xb, comb_ref[...], (((1,), (1,)), ((), ())),
        preferred_element_type=jnp.float32)
    Sc = jnp.exp(Z)

    t_blk = colid_ref[pl.ds(N + i * R, R)]
    t_col = t_blk[:, None]
    mask = colid_ref[...][None, :] == t_col

    zero = jnp.zeros_like(Sc)
    masked = jnp.where(mask, Sc, zero)
    t_y = jnp.sum(Sc[:, :N], axis=1)
    t_xh = jnp.sum(Sc[:, N:], axis=1)
    m_w = jnp.sum(masked, axis=1)
    m_xh = jnp.sum(masked[:, N:], axis=1)

    sumsq = jnp.sum(xb[:, :D] * xb[:, :D], axis=1)
    self_half = jnp.exp(TEMP * sumsq - LN2)

    a_num = m_w - self_half
    a_den = t_y + 2.0 * t_xh - m_w - m_xh

    hot = (t_col == jax.lax.broadcasted_iota(jnp.int32, (R, NTRK), 1)
           ).astype(jnp.float32)
    vals = jnp.concatenate(
        [a_num[None, :], a_den[None, :], jnp.ones((1, R), jnp.float32),
         jnp.zeros((ACC_ROWS - 3, R), jnp.float32)], axis=0)
    acc_ref[...] += jax.lax.dot_general(
        vals, hot, (((1,), (0,)), ((), ())),
        preferred_element_type=jnp.float32)

    @pl.when(i == nsteps - 1)
    def _finish():
        num = acc_ref[0, :]
        den = acc_ref[1, :]
        present = acc_ref[2, :] > 0.0
        safe_num = jnp.where(present, num, 1.0)
        safe_den = jnp.where(present, den, 1.0)
        per = jnp.where(
            present, -jnp.log(safe_num / (safe_den + safe_num)), 0.0)
        n_present = jnp.maximum(jnp.sum(present.astype(jnp.float32)), 1.0)
        out_ref[...] = (jnp.sum(per) / n_present).reshape(1)


@jax.jit
def _run(x_aug, comb, colid):
    nsteps = N // R
    return pl.pallas_call(
        _loss_kernel,
        grid=(nsteps,),
        in_specs=[
            pl.BlockSpec((R, DA), lambda i: (i, 0)),
            pl.BlockSpec((W, DA), lambda i: (0, 0)),
            pl.BlockSpec((W,), lambda i: (0,)),
        ],
        out_specs=pl.BlockSpec((1,), lambda i: (0,)),
        out_shape=jax.ShapeDtypeStruct((1,), jnp.float32),
        scratch_shapes=[pltpu.VMEM((ACC_ROWS, NTRK), jnp.float32)],
    )(x_aug, comb, colid)


def kernel(x, track_idxs, y):
    t32 = track_idxs.astype(jnp.int32)
    yf = y.reshape(-1, D)
    ones = jnp.ones((N, 1), jnp.float32)
    x_aug = jnp.concatenate([x, ones], axis=1)
    comb = jnp.concatenate(
        [jnp.concatenate([yf, jnp.zeros((N, 1), jnp.float32)], axis=1),
         jnp.concatenate([x, jnp.full((N, 1), -LN2, jnp.float32)], axis=1)],
        axis=0)
    colid = jnp.concatenate(
        [jnp.arange(N, dtype=jnp.int32) & (NTRK - 1), t32])
    return _run(x_aug, comb, colid)


# speedup vs baseline: 3.1450x; 1.1827x over previous
import jax
import jax.numpy as jnp
from jax.experimental import pallas as pl
from jax.experimental.pallas import tpu as pltpu

TEMP = 8.0
LN2 = 0.6931471805599453
LOG2E = 1.4426950408889634
N = 4096
NTRK = 512
D = 32
DA = D + 1
W = 2 * N
R = 512
ACC_ROWS = 8


def _loss_kernel(xb_ref, comb_ref, colid_ref, out_ref, acc_ref):
    i = pl.program_id(0)
    nsteps = pl.num_programs(0)

    @pl.when(i == 0)
    def _init():
        acc_ref[...] = jnp.zeros_like(acc_ref)

    fcol = jax.lax.broadcasted_iota(jnp.int32, (R, DA), 1)
    xb = jnp.where(fcol < D, xb_ref[...] * (LOG2E / TEMP), LOG2E)

    Z = jax.lax.dot_general(
        xb, comb_ref[...], (((1,), (1,)), ((), ())),
        preferred_element_type=jnp.float32)
    Sc = jnp.exp2(Z)

    t_blk = colid_ref[pl.ds(N + i * R, R)]
    t_col = t_blk[:, None]
    cid = colid_ref[...]
    mask_y = cid[None, :N] == t_col
    mask_x = cid[None, N:] == t_col

    s_y = Sc[:, :N]
    s_x = Sc[:, N:]
    t_y = jnp.sum(s_y, axis=1)
    t_xh = jnp.sum(s_x, axis=1)
    m_yh = jnp.sum(jnp.where(mask_y, s_y, 0.0), axis=1)
    m_xh = jnp.sum(jnp.where(mask_x, s_x, 0.0), axis=1)

    sumsq = jnp.sum(xb[:, :D] * xb[:, :D], axis=1)
    self_half = jnp.exp2((TEMP / LOG2E) * sumsq - 1.0)

    a_num = m_yh + m_xh - self_half
    a_den = t_y + 2.0 * t_xh - m_yh - 2.0 * m_xh

    hot = (t_col == jax.lax.broadcasted_iota(jnp.int32, (R, NTRK), 1)
           ).astype(jnp.float32)
    vals = jnp.concatenate(
        [a_num[None, :], a_den[None, :], jnp.ones((1, R), jnp.float32),
         jnp.zeros((ACC_ROWS - 3, R), jnp.float32)], axis=0)
    acc_ref[...] += jax.lax.dot_general(
        vals, hot, (((1,), (0,)), ((), ())),
        preferred_element_type=jnp.float32)

    @pl.when(i == nsteps - 1)
    def _finish():
        num = acc_ref[0, :]
        den = acc_ref[1, :]
        present = acc_ref[2, :] > 0.0
        safe_num = jnp.where(present, num, 1.0)
        safe_den = jnp.where(present, den, 1.0)
        per = jnp.where(
            present, -jnp.log(safe_num / (safe_den + safe_num)), 0.0)
        n_present = jnp.maximum(jnp.sum(present.astype(jnp.float32)), 1.0)
        out_ref[...] = (jnp.sum(per) / n_present).reshape(1)


@jax.jit
def _run(x_aug, comb, colid):
    nsteps = N // R
    return pl.pallas_call(
        _loss_kernel,
        grid=(nsteps,),
        in_specs=[
            pl.BlockSpec((R, DA), lambda i: (i, 0)),
            pl.BlockSpec((W, DA), lambda i: (0, 0)),
            pl.BlockSpec((W,), lambda i: (0,)),
        ],
        out_specs=pl.BlockSpec((1,), lambda i: (0,)),
        out_shape=jax.ShapeDtypeStruct((1,), jnp.float32),
        scratch_shapes=[pltpu.VMEM((ACC_ROWS, NTRK), jnp.float32)],
    )(x_aug, comb, colid)


def kernel(x, track_idxs, y):
    t32 = track_idxs.astype(jnp.int32)
    yf = y.reshape(-1, D)
    ones = jnp.ones((N, 1), jnp.float32)
    x_aug = jnp.concatenate([x, ones], axis=1)
    comb = jnp.concatenate(
        [jnp.concatenate([yf, jnp.zeros((N, 1), jnp.float32)], axis=1),
         jnp.concatenate([x, jnp.full((N, 1), -LN2, jnp.float32)], axis=1)],
        axis=0)
    colid = jnp.concatenate(
        [jnp.arange(N, dtype=jnp.int32) & (NTRK - 1), t32])
    return _run(x_aug, comb, colid)


# speedup vs baseline: 3.4421x; 1.0945x over previous
import jax
import jax.numpy as jnp
from jax.experimental import pallas as pl
from jax.experimental.pallas import tpu as pltpu

TEMP = 8.0
LN2 = 0.6931471805599453
LOG2E = 1.4426950408889634
N = 4096
NTRK = 512
D = 32
DA = D + 1
W = 2 * N
R = 512
ACC_ROWS = 8


def _loss_kernel(xb_ref, comb_ref, colid_ref, out_ref, acc_ref):
    i = pl.program_id(0)
    nsteps = pl.num_programs(0)

    @pl.when(i == 0)
    def _init():
        acc_ref[...] = jnp.zeros_like(acc_ref)

    fcol = jax.lax.broadcasted_iota(jnp.int32, (R, DA), 1)
    xb = jnp.where(fcol < D, xb_ref[...] * (LOG2E / TEMP), LOG2E)

    Z = jax.lax.dot_general(
        xb, comb_ref[...], (((1,), (1,)), ((), ())),
        preferred_element_type=jnp.float32)
    Sc = jnp.exp2(Z)

    t_blk = colid_ref[pl.ds(N + i * R, R)]
    t_col = t_blk[:, None]
    mask_x = colid_ref[...][None, N:] == t_col

    F = Sc[:, 0:NTRK]
    for q in range(1, N // NTRK):
        F = F + Sc[:, q * NTRK:(q + 1) * NTRK]

    s_x = Sc[:, N:]
    t_y = jnp.sum(F, axis=1)
    t_xh = jnp.sum(s_x, axis=1)
    m_xh = jnp.sum(jnp.where(mask_x, s_x, 0.0), axis=1)

    sumsq = jnp.sum(xb[:, :D] * xb[:, :D], axis=1)
    self_half = jnp.exp2((TEMP / LOG2E) * sumsq - 1.0)

    hot = (t_col == jax.lax.broadcasted_iota(jnp.int32, (R, NTRK), 1)
           ).astype(jnp.float32)
    cf = jnp.sum(hot * F, axis=0)

    a_num = m_xh - self_half
    a_den = t_y + 2.0 * t_xh - 2.0 * m_xh
    vals = jnp.concatenate(
        [a_num[None, :], a_den[None, :], jnp.ones((1, R), jnp.float32),
         jnp.zeros((ACC_ROWS - 3, R), jnp.float32)], axis=0)
    contrib = jax.lax.dot_general(
        vals, hot, (((1,), (0,)), ((), ())),
        preferred_element_type=jnp.float32)
    ridx = jax.lax.broadcasted_iota(jnp.int32, (ACC_ROWS, 1), 0)
    sgn = jnp.where(ridx == 0, 1.0,
                    jnp.where(ridx == 1, -1.0, 0.0))
    acc_ref[...] += contrib + sgn * cf[None, :]

    @pl.when(i == nsteps - 1)
    def _finish():
        num = acc_ref[0, :]
        den = acc_ref[1, :]
        present = acc_ref[2, :] > 0.0
        safe_num = jnp.where(present, num, 1.0)
        safe_den = jnp.where(present, den, 1.0)
        per = jnp.where(
            present, -jnp.log(safe_num / (safe_den + safe_num)), 0.0)
        n_present = jnp.maximum(jnp.sum(present.astype(jnp.float32)), 1.0)
        out_ref[...] = (jnp.sum(per) / n_present).reshape(1)


@jax.jit
def _run(x_aug, comb, colid):
    nsteps = N // R
    return pl.pallas_call(
        _loss_kernel,
        grid=(nsteps,),
        in_specs=[
            pl.BlockSpec((R, DA), lambda i: (i, 0)),
            pl.BlockSpec((W, DA), lambda i: (0, 0)),
            pl.BlockSpec((W,), lambda i: (0,)),
        ],
        out_specs=pl.BlockSpec((1,), lambda i: (0,)),
        out_shape=jax.ShapeDtypeStruct((1,), jnp.float32),
        scratch_shapes=[pltpu.VMEM((ACC_ROWS, NTRK), jnp.float32)],
    )(x_aug, comb, colid)


def kernel(x, track_idxs, y):
    t32 = track_idxs.astype(jnp.int32)
    yf = y.reshape(-1, D)
    ones = jnp.ones((N, 1), jnp.float32)
    x_aug = jnp.concatenate([x, ones], axis=1)
    comb = jnp.concatenate(
        [jnp.concatenate([yf, jnp.zeros((N, 1), jnp.float32)], axis=1),
         jnp.concatenate([x, jnp.full((N, 1), -LN2, jnp.float32)], axis=1)],
        axis=0)
    colid = jnp.concatenate(
        [jnp.arange(N, dtype=jnp.int32) & (NTRK - 1), t32])
    return _run(x_aug, comb, colid)


# speedup vs baseline: 3.5758x; 1.0388x over previous
import jax
import jax.numpy as jnp
from jax.experimental import pallas as pl
from jax.experimental.pallas import tpu as pltpu

TEMP = 8.0
LN2 = 0.6931471805599453
LOG2E = 1.4426950408889634
N = 4096
NTRK = 512
D = 32
DA = D + 1
W = 2 * N
R = 1024
ACC_ROWS = 8


def _loss_kernel(xb_ref, comb_ref, colid_ref, out_ref, acc_ref):
    i = pl.program_id(0)
    nsteps = pl.num_programs(0)

    @pl.when(i == 0)
    def _init():
        acc_ref[...] = jnp.zeros_like(acc_ref)

    fcol = jax.lax.broadcasted_iota(jnp.int32, (R, DA), 1)
    xb = jnp.where(fcol < D, xb_ref[...] * (LOG2E / TEMP), LOG2E)

    Z = jax.lax.dot_general(
        xb, comb_ref[...], (((1,), (1,)), ((), ())),
        preferred_element_type=jnp.float32)
    Sc = jnp.exp2(Z)

    t_blk = colid_ref[pl.ds(N + i * R, R)]
    t_col = t_blk[:, None]
    mask_x = colid_ref[...][None, N:] == t_col

    F = Sc[:, 0:NTRK]
    for q in range(1, N // NTRK):
        F = F + Sc[:, q * NTRK:(q + 1) * NTRK]

    s_x = Sc[:, N:]
    t_y = jnp.sum(F, axis=1)
    t_xh = jnp.sum(s_x, axis=1)
    m_xh = jnp.sum(jnp.where(mask_x, s_x, 0.0), axis=1)

    sumsq = jnp.sum(xb[:, :D] * xb[:, :D], axis=1)
    self_half = jnp.exp2((TEMP / LOG2E) * sumsq - 1.0)

    hot = (t_col == jax.lax.broadcasted_iota(jnp.int32, (R, NTRK), 1)
           ).astype(jnp.float32)
    cf = jnp.sum(hot * F, axis=0)

    a_num = m_xh - self_half
    a_den = t_y + 2.0 * t_xh - 2.0 * m_xh
    vals = jnp.concatenate(
        [a_num[None, :], a_den[None, :], jnp.ones((1, R), jnp.float32),
         jnp.zeros((ACC_ROWS - 3, R), jnp.float32)], axis=0)
    contrib = jax.lax.dot_general(
        vals, hot, (((1,), (0,)), ((), ())),
        preferred_element_type=jnp.float32)
    ridx = jax.lax.broadcasted_iota(jnp.int32, (ACC_ROWS, 1), 0)
    sgn = jnp.where(ridx == 0, 1.0,
                    jnp.where(ridx == 1, -1.0, 0.0))
    acc_ref[...] += contrib + sgn * cf[None, :]

    @pl.when(i == nsteps - 1)
    def _finish():
        num = acc_ref[0, :]
        den = acc_ref[1, :]
        present = acc_ref[2, :] > 0.0
        safe_num = jnp.where(present, num, 1.0)
        safe_den = jnp.where(present, den, 1.0)
        per = jnp.where(
            present, -jnp.log(safe_num / (safe_den + safe_num)), 0.0)
        n_present = jnp.maximum(jnp.sum(present.astype(jnp.float32)), 1.0)
        out_ref[...] = (jnp.sum(per) / n_present).reshape(1)


@jax.jit
def _run(x_aug, comb, colid):
    nsteps = N // R
    return pl.pallas_call(
        _loss_kernel,
        grid=(nsteps,),
        in_specs=[
            pl.BlockSpec((R, DA), lambda i: (i, 0)),
            pl.BlockSpec((W, DA), lambda i: (0, 0)),
            pl.BlockSpec((W,), lambda i: (0,)),
        ],
        out_specs=pl.BlockSpec((1,), lambda i: (0,)),
        out_shape=jax.ShapeDtypeStruct((1,), jnp.float32),
        scratch_shapes=[pltpu.VMEM((ACC_ROWS, NTRK), jnp.float32)],
    )(x_aug, comb, colid)


def kernel(x, track_idxs, y):
    t32 = track_idxs.astype(jnp.int32)
    yf = y.reshape(-1, D)
    ones = jnp.ones((N, 1), jnp.float32)
    x_aug = jnp.concatenate([x, ones], axis=1)
    comb = jnp.concatenate(
        [jnp.concatenate([yf, jnp.zeros((N, 1), jnp.float32)], axis=1),
         jnp.concatenate([x, jnp.full((N, 1), -LN2, jnp.float32)], axis=1)],
        axis=0)
    colid = jnp.concatenate(
        [jnp.arange(N, dtype=jnp.int32) & (NTRK - 1), t32])
    return _run(x_aug, comb, colid)


# speedup vs baseline: 3.7365x; 1.0449x over previous
import jax
import jax.numpy as jnp
from jax.experimental import pallas as pl
from jax.experimental.pallas import tpu as pltpu

TEMP = 8.0
LN2 = 0.6931471805599453
LOG2E = 1.4426950408889634
N = 4096
NTRK = 512
D = 32
DA = D + 1
W = 2 * N
R = 1024
ACC_ROWS = 8


def _loss_kernel(xb_ref, comb_ref, colid_ref, out_ref, acc_ref):
    i = pl.program_id(0)
    nsteps = pl.num_programs(0)

    @pl.when(i == 0)
    def _init():
        acc_ref[...] = jnp.zeros_like(acc_ref)

    fcol = jax.lax.broadcasted_iota(jnp.int32, (R, DA), 1)
    xb = jnp.where(fcol < D, xb_ref[...] * (LOG2E / TEMP), 1.0)

    Z = jax.lax.dot_general(
        xb.astype(jnp.bfloat16), comb_ref[...], (((1,), (1,)), ((), ())),
        preferred_element_type=jnp.float32)
    Sc = jnp.exp2(Z)

    t_blk = colid_ref[pl.ds(N + i * R, R)]
    t_col = t_blk[:, None]
    mask_x = colid_ref[...][None, N:] == t_col

    F = Sc[:, 0:NTRK]
    for q in range(1, N // NTRK):
        F = F + Sc[:, q * NTRK:(q + 1) * NTRK]

    s_x = Sc[:, N:]
    t_y = jnp.sum(F, axis=1)
    t_xh = jnp.sum(s_x, axis=1)
    m_xh = jnp.sum(jnp.where(mask_x, s_x, 0.0), axis=1)

    sumsq = jnp.sum(xb[:, :D] * xb[:, :D], axis=1)
    self_half = jnp.exp2((TEMP / LOG2E) * sumsq - 1.0)

    hot = (t_col == jax.lax.broadcasted_iota(jnp.int32, (R, NTRK), 1)
           ).astype(jnp.float32)
    cf = jnp.sum(hot * F, axis=0)

    a_num = m_xh - self_half
    a_den = t_y + 2.0 * t_xh - 2.0 * m_xh
    vals = jnp.concatenate(
        [a_num[None, :], a_den[None, :], jnp.ones((1, R), jnp.float32),
         jnp.zeros((ACC_ROWS - 3, R), jnp.float32)], axis=0)
    contrib = jax.lax.dot_general(
        vals, hot, (((1,), (0,)), ((), ())),
        preferred_element_type=jnp.float32)
    ridx = jax.lax.broadcasted_iota(jnp.int32, (ACC_ROWS, 1), 0)
    sgn = jnp.where(ridx == 0, 1.0,
                    jnp.where(ridx == 1, -1.0, 0.0))
    acc_ref[...] += contrib + sgn * cf[None, :]

    @pl.when(i == nsteps - 1)
    def _finish():
        num = acc_ref[0, :]
        den = acc_ref[1, :]
        present = acc_ref[2, :] > 0.0
        safe_num = jnp.where(present, num, 1.0)
        safe_den = jnp.where(present, den, 1.0)
        per = jnp.where(
            present, -jnp.log(safe_num / (safe_den + safe_num)), 0.0)
        n_present = jnp.maximum(jnp.sum(present.astype(jnp.float32)), 1.0)
        out_ref[...] = (jnp.sum(per) / n_present).reshape(1)


@jax.jit
def _run(x_aug, comb, colid):
    nsteps = N // R
    return pl.pallas_call(
        _loss_kernel,
        grid=(nsteps,),
        in_specs=[
            pl.BlockSpec((R, DA), lambda i: (i, 0)),
            pl.BlockSpec((W, DA), lambda i: (0, 0)),
            pl.BlockSpec((W,), lambda i: (0,)),
        ],
        out_specs=pl.BlockSpec((1,), lambda i: (0,)),
        out_shape=jax.ShapeDtypeStruct((1,), jnp.float32),
        scratch_shapes=[pltpu.VMEM((ACC_ROWS, NTRK), jnp.float32)],
    )(x_aug, comb, colid)


def kernel(x, track_idxs, y):
    t32 = track_idxs.astype(jnp.int32)
    yf = y.reshape(-1, D)
    ones = jnp.ones((N, 1), jnp.float32)
    x_aug = jnp.concatenate([x, ones], axis=1)
    comb = jnp.concatenate(
        [jnp.concatenate([yf, jnp.zeros((N, 1), jnp.float32)], axis=1),
         jnp.concatenate([x, jnp.full((N, 1), -1.0, jnp.float32)], axis=1)],
        axis=0).astype(jnp.bfloat16)
    colid = jnp.concatenate(
        [jnp.arange(N, dtype=jnp.int32) & (NTRK - 1), t32])
    return _run(x_aug, comb, colid)
